# baseline (speedup 1.0000x reference)
import functools

import jax
import jax.numpy as jnp
from jax import lax
from jax.experimental import pallas as pl
from jax.experimental.pallas import tpu as pltpu
from jax.experimental.pallas import tpu_sc as plsc

N_TOK = 18432
DIM = 64
K = 1024
COMMIT = 0.25

NH = N_TOK // 2
T = 1024
NT = NH // T

NC = 2
NS = 16
NW = NC * NS
BPW = NH // NW
CW = 96
CH = BPW // CW


def _dist_body(x_ref, w_ref, idx_ref, counts_ref, sumd_ref, wsq_ref):
    i = pl.program_id(0)
    w = w_ref[...]

    @pl.when(i == 0)
    def _prep():
        wsq_ref[...] = jnp.sum(w * w, axis=1)[None, :]
        sumd_ref[...] = jnp.zeros_like(sumd_ref)
        counts_ref[...] = jnp.zeros_like(counts_ref)

    x = x_ref[...]
    xsq = jnp.sum(x * x, axis=1, keepdims=True)
    w2 = w + w
    mm2 = lax.dot_general(x, w2, (((1,), (1,)), ((), ())),
                          preferred_element_type=jnp.float32)
    d = (xsq + wsq_ref[...]) - mm2
    m = jnp.min(d, axis=1, keepdims=True)
    kiota = lax.broadcasted_iota(jnp.int32, (T, K), 1)
    eq = d == m
    idx = jnp.min(jnp.where(eq, kiota, K), axis=1)
    idx_ref[0, 0, :] = idx
    onehot = (kiota == idx[:, None]).astype(jnp.float32)
    counts_ref[...] += jnp.sum(onehot, axis=0).reshape(1, K)
    sumd_ref[...] += jnp.sum(m).reshape(1, 1)


def _aux_body(w_ref, counts_ref, sumd1_ref, sumd2_ref,
              compact_ref, util_ref, quant_ref):
    w = w_ref[...]
    sq = jnp.sum(w * w, axis=1)
    g = lax.dot_general(w, w, (((1,), (1,)), ((), ())),
                        preferred_element_type=jnp.float32)
    d2 = (sq[:, None] + sq[None, :]) - 2.0 * g
    d2 = jnp.maximum(d2, 0.0)
    ri = lax.broadcasted_iota(jnp.int32, (K, K), 0)
    ci = lax.broadcasted_iota(jnp.int32, (K, K), 1)
    mask = ci > ri
    dsafe = jnp.sqrt(jnp.where(mask, d2, 1.0))
    n_pairs = K * (K - 1) // 2
    mean_pd = jnp.sum(jnp.where(mask, dsafe, 0.0)) / n_pairs
    compact_ref[...] = (2.0 * mean_pd).reshape(1, 1)
    counts = counts_ref[...]
    csum = jnp.sum(counts, axis=0, keepdims=True)
    util_ref[...] = (jnp.sum(jnp.abs(csum - N_TOK / K)) / K).reshape(1, 1)
    quant_ref[...] = ((sumd1_ref[0, 0] + sumd2_ref[0, 0])
                      * ((1.0 + COMMIT) / (N_TOK * DIM))).reshape(1, 1)


_dist_call = pl.pallas_call(
    _dist_body,
    grid=(NT,),
    in_specs=[
        pl.BlockSpec((T, DIM), lambda i: (i, 0)),
        pl.BlockSpec((K, DIM), lambda i: (0, 0)),
    ],
    out_specs=[
        pl.BlockSpec((1, 1, T), lambda i: (i, 0, 0)),
        pl.BlockSpec((1, K), lambda i: (0, 0)),
        pl.BlockSpec((1, 1), lambda i: (0, 0)),
    ],
    out_shape=[
        jax.ShapeDtypeStruct((NT, 1, T), jnp.int32),
        jax.ShapeDtypeStruct((1, K), jnp.float32),
        jax.ShapeDtypeStruct((1, 1), jnp.float32),
    ],
    scratch_shapes=[pltpu.VMEM((1, K), jnp.float32)],
    compiler_params=pltpu.CompilerParams(
        dimension_semantics=("arbitrary",)),
)

_aux_call = pl.pallas_call(
    _aux_body,
    out_shape=[
        jax.ShapeDtypeStruct((1, 1), jnp.float32),
        jax.ShapeDtypeStruct((1, 1), jnp.float32),
        jax.ShapeDtypeStruct((1, 1), jnp.float32),
    ],
)


@functools.lru_cache(maxsize=1)
def _make_sc_gather():
    mesh = plsc.VectorSubcoreMesh(core_axis_name="c", subcore_axis_name="s")

    @functools.partial(
        pl.kernel,
        mesh=mesh,
        out_type=jax.ShapeDtypeStruct((NH, DIM), jnp.float32),
        scratch_types=[
            pltpu.VMEM((CH, CW), jnp.int32),
            pltpu.VMEM((BPW, DIM), jnp.float32),
            pltpu.SemaphoreType.DMA,
        ],
        compiler_params=pltpu.CompilerParams(use_tc_tiling_on_sc=False),
    )
    def _sc_gather(w_hbm, idx2d_hbm, out_hbm, idx2_v, rows_v, sem):
        wid = lax.axis_index("s") * NC + lax.axis_index("c")
        base = wid * BPW
        pltpu.sync_copy(idx2d_hbm.at[pl.ds(wid * CH, CH)], idx2_v)
        copies = []
        for j in range(CH):
            copies.append(pltpu.async_copy(
                w_hbm.at[idx2_v.at[j]],
                rows_v.at[pl.ds(j * CW, CW)],
                sem))
        for c in copies:
            c.wait()
        pltpu.sync_copy(rows_v, out_hbm.at[pl.ds(base, BPW)])

    return _sc_gather


def kernel(x, W):
    gather = _make_sc_gather()
    idx3_1, counts1, sumd1 = _dist_call(x[:NH], W)
    q1 = gather(W, idx3_1.reshape(NH // CW, CW))
    idx3_2, counts2, sumd2 = _dist_call(x[NH:], W)
    q2 = gather(W, idx3_2.reshape(NH // CW, CW))
    counts = jnp.concatenate([counts1, counts2], axis=0)
    compact_loss, util_loss, quant_loss = _aux_call(
        W, counts, sumd1, sumd2)
    idx = jnp.concatenate(
        [idx3_1.reshape(NH), idx3_2.reshape(NH)])
    quantized = jnp.concatenate([q1, q2], axis=0)
    return (quantized, quant_loss[0, 0], util_loss[0, 0],
            compact_loss[0, 0], idx)
